# baseline (speedup 1.0000x reference)
import functools

import jax
import jax.numpy as jnp
from jax import lax
from jax.experimental import pallas as pl
from jax.experimental.pallas import tpu as pltpu
from jax.experimental.pallas import tpu_sc as plsc

NUM_USERS = 100000
HIST_LEN = 50
BATCH = 16384

_info = plsc.get_sparse_core_info()
_NC, _NS = _info.num_cores, _info.num_subcores
_NW = _NC * _NS
_B_PER_W = BATCH // _NW
_NBLK = (NUM_USERS * HIST_LEN) // 16
_NR = 4 * _B_PER_W


def _make_gather():
    mesh = plsc.VectorSubcoreMesh(core_axis_name="c", subcore_axis_name="s")

    @functools.partial(
        pl.kernel,
        mesh=mesh,
        out_type=jax.ShapeDtypeStruct((BATCH, HIST_LEN), jnp.int32),
        scratch_types=[
            pltpu.VMEM((_B_PER_W,), jnp.int32),
            pltpu.VMEM((_NR,), jnp.int32),
            pltpu.VMEM((_NR, 16), jnp.int32),
            pltpu.VMEM((_B_PER_W, HIST_LEN), jnp.int32),
            pltpu.SemaphoreType.DMA,
        ],
        compiler_params=pltpu.CompilerParams(
            use_tc_tiling_on_sc=False, needs_layout_passes=False
        ),
    )
    def gather_kernel(uid_hbm, tview_hbm, out_hbm, idx_v, gidx_v, rows_v, out_v, sem):
        wid = lax.axis_index("s") * _NC + lax.axis_index("c")
        base = wid * _B_PER_W
        pltpu.sync_copy(uid_hbm.at[pl.ds(base, _B_PER_W)], idx_v)
        iota = lax.iota(jnp.int32, 16)
        rep_sel = iota >> 2
        sub = iota & 3
        for c in range(_B_PER_W // 16):
            u = idx_v[pl.ds(c * 16, 16)]
            blk = (u * HIST_LEN) >> 4
            for q in range(4):
                rep = jnp.take(blk, rep_sel + 4 * q)
                gidx_v[pl.ds(c * 64 + q * 16, 16)] = rep + sub
        pltpu.async_copy(tview_hbm.at[gidx_v], rows_v, sem).wait()

        def chunk_body(c, carry):
            uvec = idx_v[pl.ds(c * 16, 16)]
            for l in range(16):
                i = c * 16 + l
                u = uvec[l]
                o = (u * HIST_LEN) & 15
                for ks in (0, 16, 32, 34):
                    w0 = o + ks
                    sh = w0 & 15
                    p0 = i * 4 + (w0 >> 4)
                    p1 = jnp.minimum(p0 + 1, _NR - 1)
                    a = rows_v[p0, :]
                    b = rows_v[p1, :]
                    shv = jnp.full((16,), sh, jnp.int32)
                    lane = (shv + iota) & 15
                    val = jnp.where(
                        (shv + iota) < 16, jnp.take(a, lane), jnp.take(b, lane)
                    )
                    out_v[i, pl.ds(ks, 16)] = val
            return carry

        lax.fori_loop(0, _B_PER_W // 16, chunk_body, 0)
        pltpu.sync_copy(out_v, out_hbm.at[pl.ds(base, _B_PER_W)])

    return gather_kernel


_gather = _make_gather()


def kernel(uid, table):
    tview = table.reshape(_NBLK, 16)
    return _gather(uid, tview)


# speedup vs baseline: 1.1757x; 1.1757x over previous
import functools

import jax
import jax.numpy as jnp
from jax import lax
from jax.experimental import pallas as pl
from jax.experimental.pallas import tpu as pltpu
from jax.experimental.pallas import tpu_sc as plsc

HIST_LEN = 50
BATCH = 16384

_info = plsc.get_sparse_core_info()
_NC, _NS = _info.num_cores, _info.num_subcores
_NW = _NC * _NS
_B_PER_W = BATCH // _NW
_RING = 16


def _make_gather():
    mesh = plsc.VectorSubcoreMesh(core_axis_name="c", subcore_axis_name="s")

    @functools.partial(
        pl.kernel,
        mesh=mesh,
        out_type=jax.ShapeDtypeStruct((BATCH, HIST_LEN), jnp.int32),
        scratch_types=[
            pltpu.VMEM((_B_PER_W,), jnp.int32),
            pltpu.VMEM((_RING, 8, HIST_LEN), jnp.int32),
            pltpu.VMEM((_B_PER_W, HIST_LEN), jnp.int32),
            [pltpu.SemaphoreType.DMA] * _RING,
        ],
        compiler_params=pltpu.CompilerParams(
            use_tc_tiling_on_sc=True, needs_layout_passes=False
        ),
    )
    def gather_kernel(uid_hbm, table_hbm, out_hbm, idx_v, ring_v, out_v, sems):
        wid = lax.axis_index("s") * _NC + lax.axis_index("c")
        base = wid * _B_PER_W
        pltpu.sync_copy(uid_hbm.at[pl.ds(base, _B_PER_W)], idx_v)
        iota = lax.iota(jnp.int32, 16)

        def uid_at(i):
            v = plsc.load_gather(idx_v, [iota * 0 + i])
            return v[0]

        def fire(i, slot):
            u = uid_at(i)
            r0 = pl.multiple_of((u >> 3) * 8, 8)
            return pltpu.async_copy(
                table_hbm.at[pl.ds(r0, 8)], ring_v.at[slot], sems[slot]
            )

        for s in range(_RING):
            fire(s, s)

        def body(i, carry):
            for s in range(_RING):
                @pl.when((i & (_RING - 1)) == s)
                def _():
                    pltpu.make_async_copy(
                        table_hbm.at[pl.ds(0, 8)], ring_v.at[s], sems[s]
                    ).wait()
                    u = uid_at(i)
                    r8 = u & 7
                    out_v[i, pl.ds(0, 16)] = ring_v[s, r8, pl.ds(0, 16)]
                    out_v[i, pl.ds(16, 16)] = ring_v[s, r8, pl.ds(16, 16)]
                    out_v[i, pl.ds(32, 16)] = ring_v[s, r8, pl.ds(32, 16)]
                    out_v[i, pl.ds(34, 16)] = plsc.load_gather(
                        ring_v, [iota * 0 + s, iota * 0 + r8, iota + 34]
                    )

                    @pl.when(i + _RING < _B_PER_W)
                    def _():
                        fire(i + _RING, s)
            return carry

        lax.fori_loop(0, _B_PER_W, body, 0)
        pltpu.sync_copy(out_v, out_hbm.at[pl.ds(base, _B_PER_W)])

    return gather_kernel


_gather = _make_gather()


def kernel(uid, table):
    return _gather(uid, table)


# speedup vs baseline: 1.2650x; 1.0759x over previous
import functools

import jax
import jax.numpy as jnp
from jax import lax
from jax.experimental import pallas as pl
from jax.experimental.pallas import tpu as pltpu
from jax.experimental.pallas import tpu_sc as plsc

HIST_LEN = 50
BATCH = 16384

_info = plsc.get_sparse_core_info()
_NC, _NS = _info.num_cores, _info.num_subcores
_NW = _NC * _NS
_B_PER_W = BATCH // _NW
_RING = 16


def _make_gather():
    mesh = plsc.VectorSubcoreMesh(core_axis_name="c", subcore_axis_name="s")

    @functools.partial(
        pl.kernel,
        mesh=mesh,
        out_type=jax.ShapeDtypeStruct((BATCH, HIST_LEN), jnp.int32),
        scratch_types=[
            pltpu.VMEM((_B_PER_W,), jnp.int32),
            pltpu.VMEM((_RING, 8, HIST_LEN), jnp.int32),
            pltpu.VMEM((_B_PER_W, HIST_LEN), jnp.int32),
            [pltpu.SemaphoreType.DMA] * _RING,
        ],
        compiler_params=pltpu.CompilerParams(
            use_tc_tiling_on_sc=True, needs_layout_passes=False
        ),
    )
    def gather_kernel(uid_hbm, table_hbm, out_hbm, idx_v, ring_v, out_v, sems):
        wid = lax.axis_index("s") * _NC + lax.axis_index("c")
        base = wid * _B_PER_W
        pltpu.sync_copy(uid_hbm.at[pl.ds(base, _B_PER_W)], idx_v)
        iota = lax.iota(jnp.int32, 16)

        def fire_uid(u, slot):
            r0 = pl.multiple_of((u >> 3) * 8, 8)
            return pltpu.async_copy(
                table_hbm.at[pl.ds(r0, 8)], ring_v.at[slot], sems[slot]
            )

        n_groups = _B_PER_W // 16
        uvec0 = idx_v[pl.ds(0, 16)]
        for s in range(_RING):
            fire_uid(uvec0[s], s)

        def body(gp, carry):
            uvec = idx_v[pl.ds(gp * 16, 16)]
            gnxt = jnp.minimum(gp + 1, n_groups - 1)
            uvec_next = idx_v[pl.ds(gnxt * 16, 16)]
            for half in (0, 1):
                for s in range(8):
                    slot = half * 8 + s
                    i = gp * 16 + slot
                    pltpu.make_async_copy(
                        table_hbm.at[pl.ds(0, 8)], ring_v.at[slot], sems[slot]
                    ).wait()
                    u = uvec[slot]
                    r8 = u & 7
                    out_v[i, pl.ds(0, 16)] = ring_v[slot, r8, pl.ds(0, 16)]
                    out_v[i, pl.ds(16, 16)] = ring_v[slot, r8, pl.ds(16, 16)]
                    out_v[i, pl.ds(32, 16)] = ring_v[slot, r8, pl.ds(32, 16)]
                    out_v[i, pl.ds(34, 16)] = plsc.load_gather(
                        ring_v, [iota * 0 + slot, iota * 0 + r8, iota + 34]
                    )

                @pl.when(gp + 1 < n_groups)
                def _():
                    for s in range(8):
                        slot = half * 8 + s
                        fire_uid(uvec_next[slot], slot)
            return carry

        lax.fori_loop(0, n_groups, body, 0)
        pltpu.sync_copy(out_v, out_hbm.at[pl.ds(base, _B_PER_W)])

    return gather_kernel


_gather = _make_gather()


def kernel(uid, table):
    return _gather(uid, table)
